# baseline (speedup 1.0000x reference)
import functools

import jax
import jax.numpy as jnp
from jax import lax
from jax.experimental import pallas as pl
from jax.experimental.pallas import tpu as pltpu
from jax.experimental.pallas import tpu_sc as plsc

NUM_SHARDS = 20
ROWS_PER_SHARD = 25000
EMBED_DIM = 256
VOCAB = NUM_SHARDS * ROWS_PER_SHARD
BATCH = 16384
HIST_LEN = 50

NW = 32
BPW = BATCH // NW
IDX_PER_W = BPW * HIST_LEN
CHUNK = 128
NCHUNK = IDX_PER_W // CHUNK
FIRE = 20

ROW_BLOCK = 25000

assert NCHUNK % FIRE == 0, "fire/drain loop must cover every chunk"


def _matvec_body(w_ref, t_ref, o_ref):
    res = jax.lax.dot_general(
        w_ref[...], t_ref[...],
        dimension_numbers=(((1,), (1,)), ((), ())),
        preferred_element_type=jnp.float32)
    o_ref[...] = res.reshape(1, 1, ROW_BLOCK)


def _table_times_w(table_flat, W_row):
    grid = VOCAB // ROW_BLOCK
    return pl.pallas_call(
        _matvec_body,
        grid=(grid,),
        in_specs=[
            pl.BlockSpec((1, EMBED_DIM), lambda i: (0, 0)),
            pl.BlockSpec((ROW_BLOCK, EMBED_DIM), lambda i: (i, 0)),
        ],
        out_specs=pl.BlockSpec((1, 1, ROW_BLOCK), lambda i: (i, 0, 0)),
        out_shape=jax.ShapeDtypeStruct((grid, 1, ROW_BLOCK), jnp.float32),
    )(W_row, table_flat)


def _fidx_body(i_ref, o_ref):
    v = i_ref[...]
    o_ref[...] = (v % NUM_SHARDS) * ROWS_PER_SHARD + v // NUM_SHARDS


def _flat_indices(idx3):
    return pl.pallas_call(
        _fidx_body,
        grid=(NW // 8,),
        in_specs=[pl.BlockSpec((8, NCHUNK, CHUNK), lambda i: (i, 0, 0))],
        out_specs=pl.BlockSpec((8, NCHUNK, CHUNK), lambda i: (i, 0, 0)),
        out_shape=jax.ShapeDtypeStruct((NW, NCHUNK, CHUNK), jnp.int32),
    )(idx3)


def _sc_lookup(fidx3, tv, bvec):
    mesh = plsc.VectorSubcoreMesh(core_axis_name="c", subcore_axis_name="s")

    @functools.partial(
        pl.kernel,
        mesh=mesh,
        out_type=jax.ShapeDtypeStruct((BATCH,), jnp.float32),
        scratch_types=[
            pltpu.VMEM((NCHUNK, CHUNK), jnp.int32),
            pltpu.VMEM((NCHUNK, CHUNK), jnp.float32),
            pltpu.VMEM((BPW,), jnp.float32),
            pltpu.VMEM((16,), jnp.float32),
            pltpu.VMEM_SHARED((VOCAB,), jnp.float32),
            pltpu.SemaphoreType.DMA,
        ],
    )
    def k(fidx_hbm, tv_hbm, bvec_hbm, out_hbm,
          idx_v, vals_v, pool_v, b_v, tv_sp, sem):
        cid = lax.axis_index("c")
        sid = lax.axis_index("s")
        wid = sid * 2 + cid

        pltpu.sync_copy(bvec_hbm, b_v)
        pltpu.sync_copy(fidx_hbm.at[wid], idx_v)

        @pl.when(sid == 0)
        def _():
            pltpu.sync_copy(tv_hbm, tv_sp)
        plsc.subcore_barrier()

        def fire(base):
            return [pltpu.async_copy(tv_sp.at[idx_v.at[base + i]],
                                     vals_v.at[base + i], sem)
                    for i in range(FIRE)]

        fire(0)
        def fire_drain(step, _):
            fire((step + 1) * FIRE)
            for i in range(FIRE):
                pltpu.make_async_copy(tv_sp.at[idx_v.at[step * FIRE + i]],
                                      vals_v.at[step * FIRE + i], sem).wait()
            return 0
        last = NCHUNK // FIRE - 1
        lax.fori_loop(0, last, fire_drain, 0, unroll=False)
        for i in range(FIRE):
            pltpu.make_async_copy(tv_sp.at[idx_v.at[last * FIRE + i]],
                                  vals_v.at[last * FIRE + i], sem).wait()

        bias = b_v[...]
        for g in range(BPW // 16):
            row0 = g // (CHUNK // 16)
            col = (g % (CHUNK // 16)) * 16

            def add_l(l, acc):
                return acc + vals_v[l * (BPW // CHUNK) + row0, pl.ds(col, 16)]
            acc = lax.fori_loop(0, HIST_LEN, add_l, bias, unroll=10)
            pool_v[pl.ds(g * 16, 16)] = acc

        pltpu.sync_copy(pool_v, out_hbm.at[pl.ds(wid * BPW, BPW)])

    return k(fidx3, tv, bvec)


def kernel(table, W, b, inputs):
    table_flat = table.reshape(VOCAB, EMBED_DIM)
    tv = _table_times_w(table_flat, W.reshape(1, EMBED_DIM)).reshape(VOCAB)

    idx3 = (inputs.reshape(NW, BPW, HIST_LEN)
                  .transpose(0, 2, 1)
                  .reshape(NW, NCHUNK, CHUNK))
    fidx3 = _flat_indices(idx3)

    bvec = jnp.broadcast_to(b, (16,)).astype(jnp.float32)
    pooled = _sc_lookup(fidx3, tv, bvec)
    return pooled.reshape(BATCH, 1)


# speedup vs baseline: 1.0298x; 1.0298x over previous
import functools

import jax
import jax.numpy as jnp
from jax import lax
from jax.experimental import pallas as pl
from jax.experimental.pallas import tpu as pltpu
from jax.experimental.pallas import tpu_sc as plsc

NUM_SHARDS = 20
ROWS_PER_SHARD = 25000
EMBED_DIM = 256
VOCAB = NUM_SHARDS * ROWS_PER_SHARD
BATCH = 16384
HIST_LEN = 50

NW = 32
BPW = BATCH // NW
IDX_PER_W = BPW * HIST_LEN
CHUNK = 128
NCHUNK = IDX_PER_W // CHUNK
FIRE = 20

ROW_BLOCK = 20000

assert NCHUNK % FIRE == 0, "fire/drain loop must cover every chunk"


def _matvec_body(w_ref, t_ref, o_ref):
    sub = ROW_BLOCK // 8
    w = w_ref[...]
    for k in range(8):
        res = jax.lax.dot_general(
            w, t_ref[pl.ds(k * sub, sub), :],
            dimension_numbers=(((1,), (1,)), ((), ())),
            preferred_element_type=jnp.float32)
        o_ref[0, k, :] = res[0]


def _table_times_w(table_flat, W_row):
    grid = VOCAB // ROW_BLOCK
    return pl.pallas_call(
        _matvec_body,
        grid=(grid,),
        in_specs=[
            pl.BlockSpec((1, EMBED_DIM), lambda i: (0, 0)),
            pl.BlockSpec((ROW_BLOCK, EMBED_DIM), lambda i: (i, 0)),
        ],
        out_specs=pl.BlockSpec((1, 8, ROW_BLOCK // 8), lambda i: (i, 0, 0)),
        out_shape=jax.ShapeDtypeStruct((grid, 8, ROW_BLOCK // 8), jnp.float32),
    )(W_row, table_flat)


def _fidx_body(i_ref, o_ref):
    v = i_ref[...]
    o_ref[...] = (v % NUM_SHARDS) * ROWS_PER_SHARD + v // NUM_SHARDS


def _flat_indices(idx3):
    return pl.pallas_call(
        _fidx_body,
        grid=(NW // 8,),
        in_specs=[pl.BlockSpec((8, NCHUNK, CHUNK), lambda i: (i, 0, 0))],
        out_specs=pl.BlockSpec((8, NCHUNK, CHUNK), lambda i: (i, 0, 0)),
        out_shape=jax.ShapeDtypeStruct((NW, NCHUNK, CHUNK), jnp.int32),
    )(idx3)


def _sc_lookup(fidx3, tv, bvec):
    mesh = plsc.VectorSubcoreMesh(core_axis_name="c", subcore_axis_name="s")

    @functools.partial(
        pl.kernel,
        mesh=mesh,
        out_type=jax.ShapeDtypeStruct((BATCH,), jnp.float32),
        scratch_types=[
            pltpu.VMEM((NCHUNK, CHUNK), jnp.int32),
            pltpu.VMEM((NCHUNK, CHUNK), jnp.float32),
            pltpu.VMEM((BPW,), jnp.float32),
            pltpu.VMEM((16,), jnp.float32),
            pltpu.VMEM_SHARED((VOCAB,), jnp.float32),
            pltpu.SemaphoreType.DMA,
        ],
    )
    def k(fidx_hbm, tv_hbm, bvec_hbm, out_hbm,
          idx_v, vals_v, pool_v, b_v, tv_sp, sem):
        cid = lax.axis_index("c")
        sid = lax.axis_index("s")
        wid = sid * 2 + cid

        pltpu.sync_copy(bvec_hbm, b_v)
        pltpu.sync_copy(fidx_hbm.at[wid], idx_v)

        @pl.when(sid == 0)
        def _():
            pltpu.sync_copy(tv_hbm, tv_sp)
        plsc.subcore_barrier()

        def fire(base):
            return [pltpu.async_copy(tv_sp.at[idx_v.at[base + i]],
                                     vals_v.at[base + i], sem)
                    for i in range(FIRE)]

        fire(0)
        def fire_drain(step, _):
            fire((step + 1) * FIRE)
            for i in range(FIRE):
                pltpu.make_async_copy(tv_sp.at[idx_v.at[step * FIRE + i]],
                                      vals_v.at[step * FIRE + i], sem).wait()
            return 0
        last = NCHUNK // FIRE - 1
        lax.fori_loop(0, last, fire_drain, 0, unroll=False)
        for i in range(FIRE):
            pltpu.make_async_copy(tv_sp.at[idx_v.at[last * FIRE + i]],
                                  vals_v.at[last * FIRE + i], sem).wait()

        bias = b_v[...]
        for g in range(BPW // 16):
            row0 = g // (CHUNK // 16)
            col = (g % (CHUNK // 16)) * 16

            def add_l(l, acc):
                return acc + vals_v[l * (BPW // CHUNK) + row0, pl.ds(col, 16)]
            acc = lax.fori_loop(0, HIST_LEN, add_l, bias, unroll=10)
            pool_v[pl.ds(g * 16, 16)] = acc

        pltpu.sync_copy(pool_v, out_hbm.at[pl.ds(wid * BPW, BPW)])

    return k(fidx3, tv, bvec)


def kernel(table, W, b, inputs):
    table_flat = table.reshape(VOCAB, EMBED_DIM)
    tv = _table_times_w(table_flat, W.reshape(1, EMBED_DIM)).reshape(VOCAB)

    idx3 = (inputs.reshape(NW, BPW, HIST_LEN)
                  .transpose(0, 2, 1)
                  .reshape(NW, NCHUNK, CHUNK))
    fidx3 = _flat_indices(idx3)

    bvec = jnp.broadcast_to(b, (16,)).astype(jnp.float32)
    pooled = _sc_lookup(fidx3, tv, bvec)
    return pooled.reshape(BATCH, 1)


# speedup vs baseline: 1.0308x; 1.0010x over previous
import functools

import jax
import jax.numpy as jnp
from jax import lax
from jax.experimental import pallas as pl
from jax.experimental.pallas import tpu as pltpu
from jax.experimental.pallas import tpu_sc as plsc

NUM_SHARDS = 20
ROWS_PER_SHARD = 25000
EMBED_DIM = 256
VOCAB = NUM_SHARDS * ROWS_PER_SHARD
BATCH = 16384
HIST_LEN = 50

NW = 32
BPW = BATCH // NW
IDX_PER_W = BPW * HIST_LEN
CHUNK = 128
NCHUNK = IDX_PER_W // CHUNK
FIRE = 20

ROW_BLOCK = 20000

assert NCHUNK % FIRE == 0, "fire/drain loop must cover every chunk"


MVGRID = VOCAB // ROW_BLOCK
IDX_SLAB = NCHUNK // MVGRID

assert NCHUNK % MVGRID == 0


def _matvec_body(w_ref, t_ref, i_ref, o_ref, f_ref):
    sub = ROW_BLOCK // 8
    w = w_ref[...]
    for k in range(8):
        res = jax.lax.dot_general(
            w, t_ref[pl.ds(k * sub, sub), :],
            dimension_numbers=(((1,), (1,)), ((), ())),
            preferred_element_type=jnp.float32)
        o_ref[0, k, :] = res[0]
    v = i_ref[...]
    f_ref[...] = (v % NUM_SHARDS) * ROWS_PER_SHARD + v // NUM_SHARDS


def _table_times_w_and_fidx(table_flat, W_row, idx3):
    return pl.pallas_call(
        _matvec_body,
        grid=(MVGRID,),
        in_specs=[
            pl.BlockSpec((1, EMBED_DIM), lambda i: (0, 0)),
            pl.BlockSpec((ROW_BLOCK, EMBED_DIM), lambda i: (i, 0)),
            pl.BlockSpec((NW, IDX_SLAB, CHUNK), lambda i: (0, i, 0)),
        ],
        out_specs=[
            pl.BlockSpec((1, 8, ROW_BLOCK // 8), lambda i: (i, 0, 0)),
            pl.BlockSpec((NW, IDX_SLAB, CHUNK), lambda i: (0, i, 0)),
        ],
        out_shape=[
            jax.ShapeDtypeStruct((MVGRID, 8, ROW_BLOCK // 8), jnp.float32),
            jax.ShapeDtypeStruct((NW, NCHUNK, CHUNK), jnp.int32),
        ],
    )(W_row, table_flat, idx3)


def _sc_lookup(fidx3, tv, bvec):
    mesh = plsc.VectorSubcoreMesh(core_axis_name="c", subcore_axis_name="s")

    @functools.partial(
        pl.kernel,
        mesh=mesh,
        out_type=jax.ShapeDtypeStruct((BATCH,), jnp.float32),
        scratch_types=[
            pltpu.VMEM((NCHUNK, CHUNK), jnp.int32),
            pltpu.VMEM((NCHUNK, CHUNK), jnp.float32),
            pltpu.VMEM((BPW,), jnp.float32),
            pltpu.VMEM((16,), jnp.float32),
            pltpu.VMEM_SHARED((VOCAB,), jnp.float32),
            pltpu.SemaphoreType.DMA,
        ],
    )
    def k(fidx_hbm, tv_hbm, bvec_hbm, out_hbm,
          idx_v, vals_v, pool_v, b_v, tv_sp, sem):
        cid = lax.axis_index("c")
        sid = lax.axis_index("s")
        wid = sid * 2 + cid

        pltpu.sync_copy(bvec_hbm, b_v)
        pltpu.sync_copy(fidx_hbm.at[wid], idx_v)

        @pl.when(sid == 0)
        def _():
            pltpu.sync_copy(tv_hbm, tv_sp)
        plsc.subcore_barrier()

        def fire(base):
            return [pltpu.async_copy(tv_sp.at[idx_v.at[base + i]],
                                     vals_v.at[base + i], sem)
                    for i in range(FIRE)]

        fire(0)
        def fire_drain(step, _):
            fire((step + 1) * FIRE)
            for i in range(FIRE):
                pltpu.make_async_copy(tv_sp.at[idx_v.at[step * FIRE + i]],
                                      vals_v.at[step * FIRE + i], sem).wait()
            return 0
        last = NCHUNK // FIRE - 1
        lax.fori_loop(0, last, fire_drain, 0, unroll=False)
        for i in range(FIRE):
            pltpu.make_async_copy(tv_sp.at[idx_v.at[last * FIRE + i]],
                                  vals_v.at[last * FIRE + i], sem).wait()

        bias = b_v[...]
        for g in range(BPW // 16):
            row0 = g // (CHUNK // 16)
            col = (g % (CHUNK // 16)) * 16

            def add_l(l, acc):
                return acc + vals_v[l * (BPW // CHUNK) + row0, pl.ds(col, 16)]
            acc = lax.fori_loop(0, HIST_LEN, add_l, bias, unroll=10)
            pool_v[pl.ds(g * 16, 16)] = acc

        pltpu.sync_copy(pool_v, out_hbm.at[pl.ds(wid * BPW, BPW)])

    return k(fidx3, tv, bvec)


def kernel(table, W, b, inputs):
    table_flat = table.reshape(VOCAB, EMBED_DIM)

    idx3 = (inputs.reshape(NW, BPW, HIST_LEN)
                  .transpose(0, 2, 1)
                  .reshape(NW, NCHUNK, CHUNK))
    tv3, fidx3 = _table_times_w_and_fidx(table_flat, W.reshape(1, EMBED_DIM),
                                         idx3)
    tv = tv3.reshape(VOCAB)

    bvec = jnp.broadcast_to(b, (16,)).astype(jnp.float32)
    pooled = _sc_lookup(fidx3, tv, bvec)
    return pooled.reshape(BATCH, 1)


# speedup vs baseline: 1.0325x; 1.0016x over previous
import functools

import jax
import jax.numpy as jnp
from jax import lax
from jax.experimental import pallas as pl
from jax.experimental.pallas import tpu as pltpu
from jax.experimental.pallas import tpu_sc as plsc

NUM_SHARDS = 20
ROWS_PER_SHARD = 25000
EMBED_DIM = 256
VOCAB = NUM_SHARDS * ROWS_PER_SHARD
BATCH = 16384
HIST_LEN = 50

NW = 32
BPW = BATCH // NW
IDX_PER_W = BPW * HIST_LEN
CHUNK = 128
NCHUNK = IDX_PER_W // CHUNK
FIRE = 20

ROW_BLOCK = 20000

assert NCHUNK % FIRE == 0, "fire/drain loop must cover every chunk"


MVGRID = VOCAB // ROW_BLOCK
IDX_SLAB = NCHUNK // MVGRID

assert NCHUNK % MVGRID == 0


def _matvec_body(w_ref, t_ref, i_ref, o_ref, f_ref):
    sub = ROW_BLOCK // 8
    w = w_ref[...]
    for k in range(8):
        res = jax.lax.dot_general(
            w, t_ref[pl.ds(k * sub, sub), :],
            dimension_numbers=(((1,), (1,)), ((), ())),
            preferred_element_type=jnp.float32)
        o_ref[0, k, :] = res[0]
    v = i_ref[...]
    f_ref[...] = (v % NUM_SHARDS) * ROWS_PER_SHARD + v // NUM_SHARDS


def _table_times_w_and_fidx(table_flat, W_row, idx3):
    return pl.pallas_call(
        _matvec_body,
        grid=(MVGRID,),
        in_specs=[
            pl.BlockSpec((1, EMBED_DIM), lambda i: (0, 0)),
            pl.BlockSpec((ROW_BLOCK, EMBED_DIM), lambda i: (i, 0)),
            pl.BlockSpec((NW, IDX_SLAB, CHUNK), lambda i: (0, i, 0)),
        ],
        out_specs=[
            pl.BlockSpec((1, 8, ROW_BLOCK // 8), lambda i: (i, 0, 0)),
            pl.BlockSpec((NW, IDX_SLAB, CHUNK), lambda i: (0, i, 0)),
        ],
        out_shape=[
            jax.ShapeDtypeStruct((MVGRID, 8, ROW_BLOCK // 8), jnp.float32),
            jax.ShapeDtypeStruct((NW, NCHUNK, CHUNK), jnp.int32),
        ],
    )(W_row, table_flat, idx3)


def _sc_lookup(fidx3, tv, bvec):
    mesh = plsc.VectorSubcoreMesh(core_axis_name="c", subcore_axis_name="s")

    @functools.partial(
        pl.kernel,
        mesh=mesh,
        out_type=jax.ShapeDtypeStruct((BATCH,), jnp.float32),
        scratch_types=[
            pltpu.VMEM((NCHUNK, CHUNK), jnp.int32),
            pltpu.VMEM((NCHUNK, CHUNK), jnp.float32),
            pltpu.VMEM((BPW,), jnp.float32),
            pltpu.VMEM((16,), jnp.float32),
            pltpu.VMEM((31256,), jnp.float32),
            pltpu.VMEM_SHARED((VOCAB,), jnp.float32),
            pltpu.SemaphoreType.DMA,
        ],
    )
    def k(fidx_hbm, tv_hbm, bvec_hbm, out_hbm,
          idx_v, vals_v, pool_v, b_v, stage_v, tv_sp, sem):
        cid = lax.axis_index("c")
        sid = lax.axis_index("s")
        wid = sid * 2 + cid

        pltpu.sync_copy(bvec_hbm, b_v)
        pltpu.sync_copy(fidx_hbm.at[wid], idx_v)

        slc = 31256
        tail = VOCAB - 15 * slc

        @pl.when(sid < 15)
        def _():
            pltpu.sync_copy(tv_hbm.at[pl.ds(sid * slc, slc)], stage_v)
            pltpu.sync_copy(stage_v, tv_sp.at[pl.ds(sid * slc, slc)])

        @pl.when(sid == 15)
        def _():
            pltpu.sync_copy(tv_hbm.at[pl.ds(15 * slc, tail)],
                            stage_v.at[pl.ds(0, tail)])
            pltpu.sync_copy(stage_v.at[pl.ds(0, tail)],
                            tv_sp.at[pl.ds(15 * slc, tail)])
        plsc.subcore_barrier()

        def fire(base):
            return [pltpu.async_copy(tv_sp.at[idx_v.at[base + i]],
                                     vals_v.at[base + i], sem)
                    for i in range(FIRE)]

        fire(0)
        def fire_drain(step, _):
            fire((step + 1) * FIRE)
            for i in range(FIRE):
                pltpu.make_async_copy(tv_sp.at[idx_v.at[step * FIRE + i]],
                                      vals_v.at[step * FIRE + i], sem).wait()
            return 0
        last = NCHUNK // FIRE - 1
        lax.fori_loop(0, last, fire_drain, 0, unroll=False)
        for i in range(FIRE):
            pltpu.make_async_copy(tv_sp.at[idx_v.at[last * FIRE + i]],
                                  vals_v.at[last * FIRE + i], sem).wait()

        bias = b_v[...]
        for g in range(BPW // 16):
            row0 = g // (CHUNK // 16)
            col = (g % (CHUNK // 16)) * 16

            def add_l(l, acc):
                return acc + vals_v[l * (BPW // CHUNK) + row0, pl.ds(col, 16)]
            acc = lax.fori_loop(0, HIST_LEN, add_l, bias, unroll=10)
            pool_v[pl.ds(g * 16, 16)] = acc

        pltpu.sync_copy(pool_v, out_hbm.at[pl.ds(wid * BPW, BPW)])

    return k(fidx3, tv, bvec)


def kernel(table, W, b, inputs):
    table_flat = table.reshape(VOCAB, EMBED_DIM)

    idx3 = (inputs.reshape(NW, BPW, HIST_LEN)
                  .transpose(0, 2, 1)
                  .reshape(NW, NCHUNK, CHUNK))
    tv3, fidx3 = _table_times_w_and_fidx(table_flat, W.reshape(1, EMBED_DIM),
                                         idx3)
    tv = tv3.reshape(VOCAB)

    bvec = jnp.broadcast_to(b, (16,)).astype(jnp.float32)
    pooled = _sc_lookup(fidx3, tv, bvec)
    return pooled.reshape(BATCH, 1)
